# baseline (speedup 1.0000x reference)
import dataclasses
import functools

import numpy as np
import jax
import jax.numpy as jnp
from jax import lax
from jax.experimental import pallas as pl
from jax.experimental.pallas import tpu as pltpu
from jax.experimental.pallas import tpu_sc as plsc

_D = 64
_SEQ = 16384
_NC = 2
_NS = 16
_L = 16
_NW = _NC * _NS

_TCN = 10240
_SCN = _SEQ - _TCN
_TPS = 16

_BPW = _SCN // _NW
_NGRP = _BPW // _L
_NBUF = 4


def _pe_table() -> np.ndarray:
    i = np.arange(_SEQ, dtype=np.float32)[:, None]
    j = np.arange(_D, dtype=np.float32)[None, :]
    angle = i / np.power(np.float32(10000.0), j / np.float32(_D))
    even = (np.arange(_D)[None, :] % 2) == 0
    return np.where(even, np.sin(angle), np.cos(angle)).astype(np.float32)


_PE = _pe_table()


def _compiler_params():
    cp = pltpu.CompilerParams()
    if "needs_layout_passes" in pltpu.CompilerParams.__dataclass_fields__:
        cp = dataclasses.replace(cp, needs_layout_passes=False)
    return cp


def _tc_body(xr, *refs):
    blocks, pe_ref, out_ref = refs[:_TPS], refs[_TPS], refs[_TPS + 1]
    i = pl.program_id(0)
    cols = lax.broadcasted_iota(jnp.int32, (1, 2 * _D), 1)
    for j in range(_TPS):
        o = xr[i * _TPS + j] & 127
        onehot = (cols == o).astype(jnp.float32)
        row = jax.lax.dot_general(
            onehot, blocks[j][...],
            dimension_numbers=(((1,), (1,)), ((), ())),
            preferred_element_type=jnp.float32,
        )
        out_ref[pl.ds(j, 1), :] = row + pe_ref[pl.ds(j, 1), :]


def _tc_embed(table_t, x_tc, pe_tc):
    def tab_spec(j):
        return pl.BlockSpec(
            (_D, 2 * _D),
            lambda i, xr, j=j: (0, lax.shift_right_logical(xr[i * _TPS + j], 7)),
        )

    grid_spec = pltpu.PrefetchScalarGridSpec(
        num_scalar_prefetch=1,
        grid=(_TCN // _TPS,),
        in_specs=[tab_spec(j) for j in range(_TPS)]
        + [pl.BlockSpec((_TPS, _D), lambda i, xr: (i, 0))],
        out_specs=pl.BlockSpec((_TPS, _D), lambda i, xr: (i, 0)),
    )
    return pl.pallas_call(
        _tc_body,
        grid_spec=grid_spec,
        out_shape=jax.ShapeDtypeStruct((_TCN, _D), jnp.float32),
        compiler_params=pltpu.CompilerParams(
            dimension_semantics=("arbitrary",),
        ),
    )(x_tc, *([table_t] * _TPS), pe_tc)


def _sc_embed(table_t, x_sc, pe_sc):
    mesh = plsc.VectorSubcoreMesh(core_axis_name="c", subcore_axis_name="s")

    @functools.partial(
        pl.kernel,
        out_type=jax.ShapeDtypeStruct((_SCN, _D), jnp.float32),
        mesh=mesh,
        scratch_types=[
            pltpu.VMEM((_BPW,), jnp.int32),
            pltpu.VMEM((_NBUF, _D, 2 * _D), jnp.float32),
            pltpu.VMEM((_BPW, _D), jnp.float32),
        ]
        + [pltpu.SemaphoreType.DMA] * (_NBUF + 1),
        compiler_params=_compiler_params(),
    )
    def k(tab_hbm, x_hbm, pe_hbm, out_hbm, idx_v, ring_v, pe_v, *sems):
        bsems, psem = sems[:_NBUF], sems[_NBUF]
        wid = lax.axis_index("s") * _NC + lax.axis_index("c")
        base = wid * _BPW
        pltpu.sync_copy(x_hbm.at[pl.ds(base, _BPW)], idx_v)
        pe_cp = pltpu.async_copy(pe_hbm.at[pl.ds(base, _BPW)], pe_v, psem)

        def fire(slot, xs):
            q128 = pl.multiple_of(lax.shift_right_logical(xs, 7) * 128, 128)
            pltpu.async_copy(
                tab_hbm.at[:, pl.ds(q128, 128)], ring_v.at[slot], bsems[slot]
            )

        def wait(slot):
            pltpu.make_async_copy(
                tab_hbm.at[:, pl.ds(0, 128)], ring_v.at[slot], bsems[slot]
            ).wait()

        xv0 = idx_v.at[pl.ds(0, _L)][...]
        for u in range(_NBUF):
            fire(u, xv0[u])
        pe_cp.wait()

        @pl.loop(0, _NGRP)
        def _(g):
            i0 = g * _L
            xv = idx_v.at[pl.ds(i0, _L)][...]
            nxt = jnp.minimum((g + 1) * _L, _BPW - _L)
            xn = idx_v.at[pl.ds(nxt, _L)][...]
            lanes = lax.iota(jnp.int32, _L)
            for u in range(_L):
                slot = u % _NBUF
                wait(slot)
                o = xv[u] & 127
                row = i0 + u
                for c0 in range(0, _D, _L):
                    a = plsc.load_gather(
                        ring_v.at[slot], [lanes + c0, lanes * 0 + o]
                    )
                    s = (row, pl.ds(c0, _L))
                    pe_v.at[s][...] = pe_v.at[s][...] + a
                xnext = xv[u + _NBUF] if u + _NBUF < _L else xn[u + _NBUF - _L]
                fire(slot, xnext)

        for u in range(_NBUF):
            wait(u % _NBUF)

        pltpu.sync_copy(pe_v, out_hbm.at[pl.ds(base, _BPW)])

    return k(table_t, x_sc, pe_sc)


def kernel(x, table):
    table_t = table.T
    sc_out = _sc_embed(table_t, x[_TCN:], _PE[_TCN:])
    tc_out = _tc_embed(table_t, x[:_TCN], _PE[:_TCN])
    return jnp.concatenate([tc_out, sc_out], axis=0)


# speedup vs baseline: 1.0393x; 1.0393x over previous
import dataclasses
import functools

import numpy as np
import jax
import jax.numpy as jnp
from jax import lax
from jax.experimental import pallas as pl
from jax.experimental.pallas import tpu as pltpu
from jax.experimental.pallas import tpu_sc as plsc

_D = 64
_SEQ = 16384
_NC = 2
_NS = 16
_L = 16
_NW = _NC * _NS

_TCN = 10240
_SCN = _SEQ - _TCN
_TPS = 16

_BPW = _SCN // _NW
_NGRP = _BPW // _L
_NBUF = 4


def _pe_table() -> np.ndarray:
    i = np.arange(_SEQ, dtype=np.float32)[:, None]
    j = np.arange(_D, dtype=np.float32)[None, :]
    angle = i / np.power(np.float32(10000.0), j / np.float32(_D))
    even = (np.arange(_D)[None, :] % 2) == 0
    return np.where(even, np.sin(angle), np.cos(angle)).astype(np.float32)


_PE = _pe_table()


def _compiler_params():
    cp = pltpu.CompilerParams()
    if "needs_layout_passes" in pltpu.CompilerParams.__dataclass_fields__:
        cp = dataclasses.replace(cp, needs_layout_passes=False)
    return cp


def _tc_body(xr, *refs):
    blocks, pe_ref, out_ref = refs[:_TPS], refs[_TPS], refs[_TPS + 1]
    i = pl.program_id(0)
    cols = lax.broadcasted_iota(jnp.int32, (_TPS, _TPS * 2 * _D), 1)
    tgt = jnp.stack(
        [(xr[i * _TPS + j] & 127) + j * 2 * _D for j in range(_TPS)]
    )[:, None]
    sel = (cols == tgt).astype(jnp.float32)
    b_all = jnp.concatenate([blocks[j][...] for j in range(_TPS)], axis=1)
    rows = jax.lax.dot_general(
        sel, b_all,
        dimension_numbers=(((1,), (1,)), ((), ())),
        preferred_element_type=jnp.float32,
    )
    out_ref[...] = rows + pe_ref[...]


def _tc_embed(table_t, x_tc, pe_tc):
    def tab_spec(j):
        return pl.BlockSpec(
            (_D, 2 * _D),
            lambda i, xr, j=j: (0, lax.shift_right_logical(xr[i * _TPS + j], 7)),
        )

    grid_spec = pltpu.PrefetchScalarGridSpec(
        num_scalar_prefetch=1,
        grid=(_TCN // _TPS,),
        in_specs=[tab_spec(j) for j in range(_TPS)]
        + [pl.BlockSpec((_TPS, _D), lambda i, xr: (i, 0))],
        out_specs=pl.BlockSpec((_TPS, _D), lambda i, xr: (i, 0)),
    )
    return pl.pallas_call(
        _tc_body,
        grid_spec=grid_spec,
        out_shape=jax.ShapeDtypeStruct((_TCN, _D), jnp.float32),
        compiler_params=pltpu.CompilerParams(
            dimension_semantics=("arbitrary",),
        ),
    )(x_tc, *([table_t] * _TPS), pe_tc)


def _sc_embed(table_t, x_sc, pe_sc):
    mesh = plsc.VectorSubcoreMesh(core_axis_name="c", subcore_axis_name="s")

    @functools.partial(
        pl.kernel,
        out_type=jax.ShapeDtypeStruct((_SCN, _D), jnp.float32),
        mesh=mesh,
        scratch_types=[
            pltpu.VMEM((_BPW,), jnp.int32),
            pltpu.VMEM((_NBUF, _D, 2 * _D), jnp.float32),
            pltpu.VMEM((_BPW, _D), jnp.float32),
        ]
        + [pltpu.SemaphoreType.DMA] * (_NBUF + 1),
        compiler_params=_compiler_params(),
    )
    def k(tab_hbm, x_hbm, pe_hbm, out_hbm, idx_v, ring_v, pe_v, *sems):
        bsems, psem = sems[:_NBUF], sems[_NBUF]
        wid = lax.axis_index("s") * _NC + lax.axis_index("c")
        base = wid * _BPW
        pltpu.sync_copy(x_hbm.at[pl.ds(base, _BPW)], idx_v)
        pe_cp = pltpu.async_copy(pe_hbm.at[pl.ds(base, _BPW)], pe_v, psem)

        def fire(slot, xs):
            q128 = pl.multiple_of(lax.shift_right_logical(xs, 7) * 128, 128)
            pltpu.async_copy(
                tab_hbm.at[:, pl.ds(q128, 128)], ring_v.at[slot], bsems[slot]
            )

        def wait(slot):
            pltpu.make_async_copy(
                tab_hbm.at[:, pl.ds(0, 128)], ring_v.at[slot], bsems[slot]
            ).wait()

        xv0 = idx_v.at[pl.ds(0, _L)][...]
        for u in range(_NBUF):
            fire(u, xv0[u])
        pe_cp.wait()

        @pl.loop(0, _NGRP)
        def _(g):
            i0 = g * _L
            xv = idx_v.at[pl.ds(i0, _L)][...]
            nxt = jnp.minimum((g + 1) * _L, _BPW - _L)
            xn = idx_v.at[pl.ds(nxt, _L)][...]
            lanes = lax.iota(jnp.int32, _L)
            for u in range(_L):
                slot = u % _NBUF
                wait(slot)
                o = xv[u] & 127
                row = i0 + u
                for c0 in range(0, _D, _L):
                    a = plsc.load_gather(
                        ring_v.at[slot], [lanes + c0, lanes * 0 + o]
                    )
                    s = (row, pl.ds(c0, _L))
                    pe_v.at[s][...] = pe_v.at[s][...] + a
                xnext = xv[u + _NBUF] if u + _NBUF < _L else xn[u + _NBUF - _L]
                fire(slot, xnext)

        for u in range(_NBUF):
            wait(u % _NBUF)

        pltpu.sync_copy(pe_v, out_hbm.at[pl.ds(base, _BPW)])

    return k(table_t, x_sc, pe_sc)


def kernel(x, table):
    table_t = table.T
    sc_out = _sc_embed(table_t, x[_TCN:], _PE[_TCN:])
    tc_out = _tc_embed(table_t, x[:_TCN], _PE[:_TCN])
    return jnp.concatenate([tc_out, sc_out], axis=0)


# speedup vs baseline: 1.2186x; 1.1725x over previous
import dataclasses
import functools

import numpy as np
import jax
import jax.numpy as jnp
from jax import lax
from jax.experimental import pallas as pl
from jax.experimental.pallas import tpu as pltpu
from jax.experimental.pallas import tpu_sc as plsc

_D = 64
_SEQ = 16384
_NC = 2
_NS = 16
_L = 16
_NW = _NC * _NS

_TCN = 10240
_SCN = _SEQ - _TCN
_TPS = 16

_BPW = _SCN // _NW
_NGRP = _BPW // _L
_NBUF = 4


def _pe_table() -> np.ndarray:
    i = np.arange(_SEQ, dtype=np.float32)[:, None]
    j = np.arange(_D, dtype=np.float32)[None, :]
    angle = i / np.power(np.float32(10000.0), j / np.float32(_D))
    even = (np.arange(_D)[None, :] % 2) == 0
    return np.where(even, np.sin(angle), np.cos(angle)).astype(np.float32)


_PE = _pe_table()


def _compiler_params():
    cp = pltpu.CompilerParams()
    if "needs_layout_passes" in pltpu.CompilerParams.__dataclass_fields__:
        cp = dataclasses.replace(cp, needs_layout_passes=False)
    return cp


def _tc_body(xr, tab_hbm, pe_ref, out_ref, ring, sems):
    def fire(half, u, k):
        q = lax.shift_right_logical(xr[k], 7)
        pltpu.make_async_copy(
            tab_hbm.at[:, pl.ds(pl.multiple_of(q * 2 * _D, 2 * _D), 2 * _D)],
            ring.at[half, u],
            sems.at[half, u],
        ).start()

    def wait(half, u):
        pltpu.make_async_copy(
            tab_hbm.at[:, pl.ds(0, 2 * _D)], ring.at[half, u], sems.at[half, u]
        ).wait()

    for u in range(_TPS):
        fire(0, u, u)
    for u in range(_TPS):
        fire(1, u, _TPS + u)

    cols = lax.broadcasted_iota(jnp.int32, (_TPS, _TPS * 2 * _D), 1)

    @pl.loop(0, _TCN // _TPS)
    def _(g):
        half = g % 2
        for u in range(_TPS):
            wait(half, u)
        b_all = jnp.concatenate(
            [ring.at[half, u][...] for u in range(_TPS)], axis=1
        )
        tgt = jnp.stack(
            [(xr[g * _TPS + u] & 127) + u * 2 * _D for u in range(_TPS)]
        )[:, None]
        sel = (cols == tgt).astype(jnp.float32)
        rows = jax.lax.dot_general(
            sel, b_all,
            dimension_numbers=(((1,), (1,)), ((), ())),
            preferred_element_type=jnp.float32,
        )
        s = pl.ds(g * _TPS, _TPS)
        out_ref[s, :] = rows + pe_ref[s, :]
        for u in range(_TPS):
            k2 = jnp.minimum((g + 2) * _TPS + u, _TCN - 1)
            fire(half, u, k2)

    for half in range(2):
        for u in range(_TPS):
            wait(half, u)


def _tc_embed(table_t, x_tc, pe_tc):
    grid_spec = pltpu.PrefetchScalarGridSpec(
        num_scalar_prefetch=1,
        grid=(1,),
        in_specs=[
            pl.BlockSpec(memory_space=pl.ANY),
            pl.BlockSpec((_TCN, _D), lambda i, xr: (0, 0)),
        ],
        out_specs=pl.BlockSpec((_TCN, _D), lambda i, xr: (0, 0)),
        scratch_shapes=[
            pltpu.VMEM((2, _TPS, _D, 2 * _D), jnp.float32),
            pltpu.SemaphoreType.DMA((2, _TPS)),
        ],
    )
    return pl.pallas_call(
        _tc_body,
        grid_spec=grid_spec,
        out_shape=jax.ShapeDtypeStruct((_TCN, _D), jnp.float32),
        compiler_params=pltpu.CompilerParams(
            dimension_semantics=("arbitrary",),
        ),
    )(x_tc, table_t, pe_tc)


def _sc_embed(table_t, x_sc, pe_sc):
    mesh = plsc.VectorSubcoreMesh(core_axis_name="c", subcore_axis_name="s")

    @functools.partial(
        pl.kernel,
        out_type=jax.ShapeDtypeStruct((_SCN, _D), jnp.float32),
        mesh=mesh,
        scratch_types=[
            pltpu.VMEM((_BPW,), jnp.int32),
            pltpu.VMEM((_NBUF, _D, 2 * _D), jnp.float32),
            pltpu.VMEM((_BPW, _D), jnp.float32),
        ]
        + [pltpu.SemaphoreType.DMA] * (_NBUF + 1),
        compiler_params=_compiler_params(),
    )
    def k(tab_hbm, x_hbm, pe_hbm, out_hbm, idx_v, ring_v, pe_v, *sems):
        bsems, psem = sems[:_NBUF], sems[_NBUF]
        wid = lax.axis_index("s") * _NC + lax.axis_index("c")
        base = wid * _BPW
        pltpu.sync_copy(x_hbm.at[pl.ds(base, _BPW)], idx_v)
        pe_cp = pltpu.async_copy(pe_hbm.at[pl.ds(base, _BPW)], pe_v, psem)

        def fire(slot, xs):
            q128 = pl.multiple_of(lax.shift_right_logical(xs, 7) * 128, 128)
            pltpu.async_copy(
                tab_hbm.at[:, pl.ds(q128, 128)], ring_v.at[slot], bsems[slot]
            )

        def wait(slot):
            pltpu.make_async_copy(
                tab_hbm.at[:, pl.ds(0, 128)], ring_v.at[slot], bsems[slot]
            ).wait()

        xv0 = idx_v.at[pl.ds(0, _L)][...]
        for u in range(_NBUF):
            fire(u, xv0[u])
        pe_cp.wait()

        @pl.loop(0, _NGRP)
        def _(g):
            i0 = g * _L
            xv = idx_v.at[pl.ds(i0, _L)][...]
            nxt = jnp.minimum((g + 1) * _L, _BPW - _L)
            xn = idx_v.at[pl.ds(nxt, _L)][...]
            lanes = lax.iota(jnp.int32, _L)
            for u in range(_L):
                slot = u % _NBUF
                wait(slot)
                o = xv[u] & 127
                row = i0 + u
                for c0 in range(0, _D, _L):
                    a = plsc.load_gather(
                        ring_v.at[slot], [lanes + c0, lanes * 0 + o]
                    )
                    s = (row, pl.ds(c0, _L))
                    pe_v.at[s][...] = pe_v.at[s][...] + a
                xnext = xv[u + _NBUF] if u + _NBUF < _L else xn[u + _NBUF - _L]
                fire(slot, xnext)

        for u in range(_NBUF):
            wait(u % _NBUF)

        pltpu.sync_copy(pe_v, out_hbm.at[pl.ds(base, _BPW)])

    return k(table_t, x_sc, pe_sc)


def kernel(x, table):
    table_t = table.T
    sc_out = _sc_embed(table_t, x[_TCN:], _PE[_TCN:])
    tc_out = _tc_embed(table_t, x[:_TCN], _PE[:_TCN])
    return jnp.concatenate([tc_out, sc_out], axis=0)


# speedup vs baseline: 2.1019x; 1.7248x over previous
import dataclasses
import functools

import numpy as np
import jax
import jax.numpy as jnp
from jax import lax
from jax.experimental import pallas as pl
from jax.experimental.pallas import tpu as pltpu
from jax.experimental.pallas import tpu_sc as plsc

_D = 64
_SEQ = 16384
_NC = 2
_NS = 16
_L = 16
_NW = _NC * _NS

_TCN = 4608
_SCN = _SEQ - _TCN
_TPS = 16

_BPW = _SCN // _NW
_NGRP = _BPW // _L
_NBUF = 4


def _pe_table() -> np.ndarray:
    i = np.arange(_SEQ, dtype=np.float32)[:, None]
    j = np.arange(_D, dtype=np.float32)[None, :]
    angle = i / np.power(np.float32(10000.0), j / np.float32(_D))
    even = (np.arange(_D)[None, :] % 2) == 0
    return np.where(even, np.sin(angle), np.cos(angle)).astype(np.float32)


_PE = _pe_table()


def _compiler_params():
    cp = pltpu.CompilerParams()
    if "needs_layout_passes" in pltpu.CompilerParams.__dataclass_fields__:
        cp = dataclasses.replace(cp, needs_layout_passes=False)
    return cp


def _tc_body(xr, tab_hbm, pe_ref, out_ref, ring, sems):
    def fire(half, u, k):
        q = lax.shift_right_logical(xr[k], 7)
        pltpu.make_async_copy(
            tab_hbm.at[:, pl.ds(pl.multiple_of(q * 2 * _D, 2 * _D), 2 * _D)],
            ring.at[half, u],
            sems.at[half, u],
        ).start()

    def wait(half, u):
        pltpu.make_async_copy(
            tab_hbm.at[:, pl.ds(0, 2 * _D)], ring.at[half, u], sems.at[half, u]
        ).wait()

    for u in range(_TPS):
        fire(0, u, u)
    for u in range(_TPS):
        fire(1, u, _TPS + u)

    cols = lax.broadcasted_iota(jnp.int32, (_TPS, _TPS * 2 * _D), 1)

    @pl.loop(0, _TCN // _TPS)
    def _(g):
        half = g % 2
        for u in range(_TPS):
            wait(half, u)
        b_all = jnp.concatenate(
            [ring.at[half, u][...] for u in range(_TPS)], axis=1
        )
        tgt = jnp.stack(
            [(xr[g * _TPS + u] & 127) + u * 2 * _D for u in range(_TPS)]
        )[:, None]
        sel = (cols == tgt).astype(jnp.float32)
        rows = jax.lax.dot_general(
            sel, b_all,
            dimension_numbers=(((1,), (1,)), ((), ())),
            preferred_element_type=jnp.float32,
        )
        s = pl.ds(g * _TPS, _TPS)
        out_ref[s, :] = rows + pe_ref[s, :]
        for u in range(_TPS):
            k2 = jnp.minimum((g + 2) * _TPS + u, _TCN - 1)
            fire(half, u, k2)

    for half in range(2):
        for u in range(_TPS):
            wait(half, u)


def _tc_embed(table_t, x_tc, pe_tc):
    grid_spec = pltpu.PrefetchScalarGridSpec(
        num_scalar_prefetch=1,
        grid=(1,),
        in_specs=[
            pl.BlockSpec(memory_space=pl.ANY),
            pl.BlockSpec((_TCN, _D), lambda i, xr: (0, 0)),
        ],
        out_specs=pl.BlockSpec((_TCN, _D), lambda i, xr: (0, 0)),
        scratch_shapes=[
            pltpu.VMEM((2, _TPS, _D, 2 * _D), jnp.float32),
            pltpu.SemaphoreType.DMA((2, _TPS)),
        ],
    )
    return pl.pallas_call(
        _tc_body,
        grid_spec=grid_spec,
        out_shape=jax.ShapeDtypeStruct((_TCN, _D), jnp.float32),
        compiler_params=pltpu.CompilerParams(
            dimension_semantics=("arbitrary",),
        ),
    )(x_tc, table_t, pe_tc)


def _sc_embed(table_t, x_sc, pe_sc):
    mesh = plsc.VectorSubcoreMesh(core_axis_name="c", subcore_axis_name="s")

    @functools.partial(
        pl.kernel,
        out_type=jax.ShapeDtypeStruct((_SCN, _D), jnp.float32),
        mesh=mesh,
        scratch_types=[
            pltpu.VMEM((_BPW,), jnp.int32),
            pltpu.VMEM((_NBUF, _D, 2 * _D), jnp.float32),
            pltpu.VMEM((_BPW, _D), jnp.float32),
        ]
        + [pltpu.SemaphoreType.DMA] * (_NBUF + 1),
        compiler_params=_compiler_params(),
    )
    def k(tab_hbm, x_hbm, pe_hbm, out_hbm, idx_v, ring_v, pe_v, *sems):
        bsems, psem = sems[:_NBUF], sems[_NBUF]
        wid = lax.axis_index("s") * _NC + lax.axis_index("c")
        base = wid * _BPW
        pltpu.sync_copy(x_hbm.at[pl.ds(base, _BPW)], idx_v)
        pe_cp = pltpu.async_copy(pe_hbm.at[pl.ds(base, _BPW)], pe_v, psem)

        def fire(slot, xs):
            q128 = pl.multiple_of(lax.shift_right_logical(xs, 7) * 128, 128)
            pltpu.async_copy(
                tab_hbm.at[:, pl.ds(q128, 128)], ring_v.at[slot], bsems[slot]
            )

        def wait(slot):
            pltpu.make_async_copy(
                tab_hbm.at[:, pl.ds(0, 128)], ring_v.at[slot], bsems[slot]
            ).wait()

        xv0 = idx_v.at[pl.ds(0, _L)][...]
        for u in range(_NBUF):
            fire(u, xv0[u])
        pe_cp.wait()

        @pl.loop(0, _NGRP)
        def _(g):
            i0 = g * _L
            xv = idx_v.at[pl.ds(i0, _L)][...]
            nxt = jnp.minimum((g + 1) * _L, _BPW - _L)
            xn = idx_v.at[pl.ds(nxt, _L)][...]
            lanes = lax.iota(jnp.int32, _L)
            for u in range(_L):
                slot = u % _NBUF
                wait(slot)
                o = xv[u] & 127
                row = i0 + u
                for c0 in range(0, _D, _L):
                    a = plsc.load_gather(
                        ring_v.at[slot], [lanes + c0, lanes * 0 + o]
                    )
                    s = (row, pl.ds(c0, _L))
                    pe_v.at[s][...] = pe_v.at[s][...] + a
                xnext = xv[u + _NBUF] if u + _NBUF < _L else xn[u + _NBUF - _L]
                fire(slot, xnext)

        for u in range(_NBUF):
            wait(u % _NBUF)

        pltpu.sync_copy(pe_v, out_hbm.at[pl.ds(base, _BPW)])

    return k(table_t, x_sc, pe_sc)


def kernel(x, table):
    table_t = table.T
    sc_out = _sc_embed(table_t, x[_TCN:], _PE[_TCN:])
    tc_out = _tc_embed(table_t, x[:_TCN], _PE[:_TCN])
    return jnp.concatenate([tc_out, sc_out], axis=0)


# speedup vs baseline: 2.3005x; 1.0945x over previous
import dataclasses
import functools

import numpy as np
import jax
import jax.numpy as jnp
from jax import lax
from jax.experimental import pallas as pl
from jax.experimental.pallas import tpu as pltpu
from jax.experimental.pallas import tpu_sc as plsc

_D = 64
_SEQ = 16384
_NC = 2
_NS = 16
_L = 16
_NW = _NC * _NS

_TCN = 3072
_SCN = _SEQ - _TCN
_TPS = 16

_BPW = _SCN // _NW
_NGRP = _BPW // _L
_NBUF = 4


def _pe_table() -> np.ndarray:
    i = np.arange(_SEQ, dtype=np.float32)[:, None]
    j = np.arange(_D, dtype=np.float32)[None, :]
    angle = i / np.power(np.float32(10000.0), j / np.float32(_D))
    even = (np.arange(_D)[None, :] % 2) == 0
    return np.where(even, np.sin(angle), np.cos(angle)).astype(np.float32)


_PE = _pe_table()


def _compiler_params():
    cp = pltpu.CompilerParams()
    if "needs_layout_passes" in pltpu.CompilerParams.__dataclass_fields__:
        cp = dataclasses.replace(cp, needs_layout_passes=False)
    return cp


def _tc_body(xr, tab_hbm, pe_ref, out_ref, ring, sems):
    def fire(half, u, k):
        q = lax.shift_right_logical(xr[k], 7)
        pltpu.make_async_copy(
            tab_hbm.at[:, pl.ds(pl.multiple_of(q * 2 * _D, 2 * _D), 2 * _D)],
            ring.at[half, u],
            sems.at[half, u],
        ).start()

    def wait(half, u):
        pltpu.make_async_copy(
            tab_hbm.at[:, pl.ds(0, 2 * _D)], ring.at[half, u], sems.at[half, u]
        ).wait()

    for u in range(_TPS):
        fire(0, u, u)
    for u in range(_TPS):
        fire(1, u, _TPS + u)

    cols = lax.broadcasted_iota(jnp.int32, (_TPS, _TPS * 2 * _D), 1)

    @pl.loop(0, _TCN // _TPS)
    def _(g):
        half = g % 2
        for u in range(_TPS):
            wait(half, u)
        b_all = jnp.concatenate(
            [ring.at[half, u][...] for u in range(_TPS)], axis=1
        )
        tgt = jnp.stack(
            [(xr[g * _TPS + u] & 127) + u * 2 * _D for u in range(_TPS)]
        )[:, None]
        sel = (cols == tgt).astype(jnp.float32)
        rows = jax.lax.dot_general(
            sel, b_all,
            dimension_numbers=(((1,), (1,)), ((), ())),
            preferred_element_type=jnp.float32,
        )
        s = pl.ds(g * _TPS, _TPS)
        out_ref[s, :] = rows + pe_ref[s, :]
        for u in range(_TPS):
            k2 = jnp.minimum((g + 2) * _TPS + u, _TCN - 1)
            fire(half, u, k2)

    for half in range(2):
        for u in range(_TPS):
            wait(half, u)


def _tc_embed(table_t, x_tc, pe_tc):
    grid_spec = pltpu.PrefetchScalarGridSpec(
        num_scalar_prefetch=1,
        grid=(1,),
        in_specs=[
            pl.BlockSpec(memory_space=pl.ANY),
            pl.BlockSpec((_TCN, _D), lambda i, xr: (0, 0)),
        ],
        out_specs=pl.BlockSpec((_TCN, _D), lambda i, xr: (0, 0)),
        scratch_shapes=[
            pltpu.VMEM((2, _TPS, _D, 2 * _D), jnp.float32),
            pltpu.SemaphoreType.DMA((2, _TPS)),
        ],
    )
    return pl.pallas_call(
        _tc_body,
        grid_spec=grid_spec,
        out_shape=jax.ShapeDtypeStruct((_TCN, _D), jnp.float32),
        compiler_params=pltpu.CompilerParams(
            dimension_semantics=("arbitrary",),
        ),
    )(x_tc, table_t, pe_tc)


def _sc_embed(table_t, x_sc, pe_sc):
    mesh = plsc.VectorSubcoreMesh(core_axis_name="c", subcore_axis_name="s")

    @functools.partial(
        pl.kernel,
        out_type=jax.ShapeDtypeStruct((_SCN, _D), jnp.float32),
        mesh=mesh,
        scratch_types=[
            pltpu.VMEM((_BPW,), jnp.int32),
            pltpu.VMEM((_NBUF, _D, 2 * _D), jnp.float32),
            pltpu.VMEM((_BPW, _D), jnp.float32),
        ]
        + [pltpu.SemaphoreType.DMA] * (_NBUF + 1),
        compiler_params=_compiler_params(),
    )
    def k(tab_hbm, x_hbm, pe_hbm, out_hbm, idx_v, ring_v, pe_v, *sems):
        bsems, psem = sems[:_NBUF], sems[_NBUF]
        wid = lax.axis_index("s") * _NC + lax.axis_index("c")
        base = wid * _BPW
        pltpu.sync_copy(x_hbm.at[pl.ds(base, _BPW)], idx_v)
        pe_cp = pltpu.async_copy(pe_hbm.at[pl.ds(base, _BPW)], pe_v, psem)

        def fire(slot, xs):
            q128 = pl.multiple_of(lax.shift_right_logical(xs, 7) * 128, 128)
            pltpu.async_copy(
                tab_hbm.at[:, pl.ds(q128, 128)], ring_v.at[slot], bsems[slot]
            )

        def wait(slot):
            pltpu.make_async_copy(
                tab_hbm.at[:, pl.ds(0, 128)], ring_v.at[slot], bsems[slot]
            ).wait()

        xv0 = idx_v.at[pl.ds(0, _L)][...]
        for u in range(_NBUF):
            fire(u, xv0[u])
        pe_cp.wait()

        @pl.loop(0, _NGRP)
        def _(g):
            i0 = g * _L
            xv = idx_v.at[pl.ds(i0, _L)][...]
            nxt = jnp.minimum((g + 1) * _L, _BPW - _L)
            xn = idx_v.at[pl.ds(nxt, _L)][...]
            lanes = lax.iota(jnp.int32, _L)
            for u in range(_L):
                slot = u % _NBUF
                wait(slot)
                o = xv[u] & 127
                row = i0 + u
                for c0 in range(0, _D, _L):
                    a = plsc.load_gather(
                        ring_v.at[slot], [lanes + c0, lanes * 0 + o]
                    )
                    s = (row, pl.ds(c0, _L))
                    pe_v.at[s][...] = pe_v.at[s][...] + a
                xnext = xv[u + _NBUF] if u + _NBUF < _L else xn[u + _NBUF - _L]
                fire(slot, xnext)

        for u in range(_NBUF):
            wait(u % _NBUF)

        pltpu.sync_copy(pe_v, out_hbm.at[pl.ds(base, _BPW)])

    return k(table_t, x_sc, pe_sc)


def kernel(x, table):
    table_t = table.T
    sc_out = _sc_embed(table_t, x[_TCN:], _PE[_TCN:])
    tc_out = _tc_embed(table_t, x[:_TCN], _PE[:_TCN])
    return jnp.concatenate([tc_out, sc_out], axis=0)
